# baseline (speedup 1.0000x reference)
import jax
import jax.numpy as jnp
from jax import lax
from jax.experimental import pallas as pl
from jax.experimental.pallas import tpu as pltpu
from jax.experimental.pallas import tpu_sc as plsc

N = 10000
E = 320000
D_IN = 128
D_HID = 64
D_OUT = 40

NC = 2
NS = 16
NW = NC * NS
NPAD = 10112
ROWS_PER_TILE = NPAD // NS

EPW = E // NW
VB = EPW // 16

_f32 = jnp.float32


def _sc_mesh():
    return plsc.VectorSubcoreMesh(core_axis_name="c", subcore_axis_name="s")


def _deg_body(dst_hbm, out_hbm, dst_v, deg_v):
    c = lax.axis_index("c")
    s = lax.axis_index("s")
    wid = s * NC + c
    pltpu.sync_copy(dst_hbm.at[pl.ds(wid * EPW, EPW)], dst_v)

    def zero_body(i, carry):
        deg_v[pl.ds(i * 16, 16)] = jnp.zeros((16,), _f32)
        return carry

    lax.fori_loop(0, EPW // 16, zero_body, 0)

    ones = jnp.full((16,), 1.0, _f32)

    def acc_body(i, carry):
        idx = dst_v[pl.ds(i * 16, 16)]
        plsc.addupdate_scatter(deg_v, [idx], ones)
        return carry

    lax.fori_loop(0, VB, acc_body, 0)
    pltpu.sync_copy(deg_v, out_hbm.at[wid])


_deg_kernel = pl.kernel(
    _deg_body,
    out_type=jax.ShapeDtypeStruct((NW, EPW), _f32),
    mesh=_sc_mesh(),
    compiler_params=pltpu.CompilerParams(needs_layout_passes=False),
    scratch_types=[
        pltpu.VMEM((EPW,), jnp.int32),
        pltpu.VMEM((EPW,), _f32),
    ],
)


CHUNK = 256
NCHUNKS = 1280
N0 = NCHUNKS // NS
NMAX = N0
EPAD = NCHUNKS * CHUNK


def _make_agg(d):
    def body(z_hbm, src_hbm, dst_hbm, out_hbm, src_v, dst_v,
             rows_a, rows_b, acc, sem_a, sem_b):
        c = lax.axis_index("c")
        s = lax.axis_index("s")
        row0 = s * ROWS_PER_TILE

        @pl.when(c == 0)
        def _():
            start = s * N0
            pltpu.sync_copy(src_hbm.at[pl.ds(start, NMAX)], src_v)
            pltpu.sync_copy(dst_hbm.at[pl.ds(start, NMAX)], dst_v)
            pltpu.sync_copy(z_hbm.at[pl.ds(row0, ROWS_PER_TILE)],
                            acc.at[pl.ds(row0, ROWS_PER_TILE)])
            plsc.subcore_barrier()

            def step(i, carry):
                j = 2 * i
                ga = pltpu.async_copy(z_hbm.at[src_v.at[j]], rows_a, sem_a)
                gb = pltpu.async_copy(z_hbm.at[src_v.at[j + 1]], rows_b, sem_b)
                ga.wait()
                pltpu.sync_copy(rows_a, acc.at[dst_v.at[j]], add=True)
                gb.wait()
                pltpu.sync_copy(rows_b, acc.at[dst_v.at[j + 1]], add=True)
                return carry

            lax.fori_loop(0, N0 // 2, step, 0)
            plsc.subcore_barrier()
            pltpu.sync_copy(acc.at[pl.ds(row0, ROWS_PER_TILE)],
                            out_hbm.at[pl.ds(row0, ROWS_PER_TILE)])

    return pl.kernel(
        body,
        out_type=jax.ShapeDtypeStruct((NPAD, d), _f32),
        mesh=_sc_mesh(),
        compiler_params=pltpu.CompilerParams(use_tc_tiling_on_sc=False),
        scratch_types=[
            pltpu.VMEM((NMAX, CHUNK), jnp.int32),
            pltpu.VMEM((NMAX, CHUNK), jnp.int32),
            pltpu.VMEM((CHUNK, d), _f32),
            pltpu.VMEM((CHUNK, d), _f32),
            pltpu.VMEM_SHARED((NPAD, d), _f32),
            pltpu.SemaphoreType.DMA,
            pltpu.SemaphoreType.DMA,
        ],
    )


_agg_hid = _make_agg(D_HID)
_agg_out = _make_agg(D_OUT)


def _tc_a_body(deg_ref, x_ref, w_ref, z_ref, dinv_ref):
    deg_parts = deg_ref[...]
    ones = jnp.ones((NW, 1), _f32)
    deg = lax.dot_general(deg_parts, ones, (((0,), (0,)), ((), ())),
                          preferred_element_type=_f32)
    dinv = lax.rsqrt(deg + 1.0)
    dinv_pad = jnp.concatenate(
        [dinv, jnp.zeros((NPAD - N, 1), _f32)], axis=0)
    z = jnp.dot(x_ref[...], w_ref[...], preferred_element_type=_f32)
    z_ref[...] = z * dinv_pad
    dinv_ref[...] = dinv_pad


_tc_a = pl.pallas_call(
    _tc_a_body,
    out_shape=[
        jax.ShapeDtypeStruct((NPAD, D_HID), _f32),
        jax.ShapeDtypeStruct((NPAD, 1), _f32),
    ],
)


def _tc_b_body(p_ref, dinv_ref, b1_ref, w2_ref, z2_ref):
    agg = p_ref[...]
    dinv = dinv_ref[...]
    h = jnp.maximum(dinv * agg + b1_ref[...], 0.0)
    z2 = jnp.dot(h, w2_ref[...], preferred_element_type=_f32)
    z2_ref[...] = z2 * dinv


_tc_b = pl.pallas_call(
    _tc_b_body,
    out_shape=jax.ShapeDtypeStruct((NPAD, D_OUT), _f32),
)


def _tc_c_body(q_ref, dinv_ref, b2_ref, out_ref):
    agg = q_ref[...]
    o = dinv_ref[...] * agg + b2_ref[...]
    m = jnp.max(o, axis=1, keepdims=True)
    sh = o - m
    lse = jnp.log(jnp.sum(jnp.exp(sh), axis=1, keepdims=True))
    out_ref[...] = sh - lse


_tc_c = pl.pallas_call(
    _tc_c_body,
    out_shape=jax.ShapeDtypeStruct((NPAD, D_OUT), _f32),
)


def kernel(x, edge_index, W1, b1, W2, b2):
    src = edge_index[0]
    dst = edge_index[1]

    deg_parts = _deg_kernel(dst)

    x_pad = jnp.concatenate(
        [x, jnp.zeros((NPAD - N, D_IN), _f32)], axis=0)
    z1, dinv = _tc_a(deg_parts, x_pad, W1)

    pad_idx = jnp.full((EPAD - E,), N, jnp.int32)
    src_p = jnp.concatenate([src, pad_idx]).reshape(-1, CHUNK)
    dst_p = jnp.concatenate([dst, pad_idx]).reshape(-1, CHUNK)

    p = _agg_hid(z1, src_p, dst_p)
    z2 = _tc_b(p, dinv, b1.reshape(1, D_HID), W2)
    q = _agg_out(z2, src_p, dst_p)
    out = _tc_c(q, dinv, b2.reshape(1, D_OUT))
    return out[:N]


# speedup vs baseline: 1.4904x; 1.4904x over previous
import jax
import jax.numpy as jnp
from jax import lax
from jax.experimental import pallas as pl
from jax.experimental.pallas import tpu as pltpu
from jax.experimental.pallas import tpu_sc as plsc

N = 10000
E = 320000
D_IN = 128
D_HID = 64
D_OUT = 40

NC = 2
NS = 16
NW = NC * NS
NPAD = 10112
ROWS_PER_TILE = NPAD // NS

EPW = E // NW
VB = EPW // 16

_f32 = jnp.float32


def _sc_mesh():
    return plsc.VectorSubcoreMesh(core_axis_name="c", subcore_axis_name="s")


def _deg_body(dst_hbm, out_hbm, dst_v, deg_v):
    c = lax.axis_index("c")
    s = lax.axis_index("s")
    wid = s * NC + c
    pltpu.sync_copy(dst_hbm.at[pl.ds(wid * EPW, EPW)], dst_v)

    def zero_body(i, carry):
        deg_v[pl.ds(i * 16, 16)] = jnp.zeros((16,), _f32)
        return carry

    lax.fori_loop(0, EPW // 16, zero_body, 0)

    ones = jnp.full((16,), 1.0, _f32)

    def acc_body(i, carry):
        idx = dst_v[pl.ds(i * 16, 16)]
        plsc.addupdate_scatter(deg_v, [idx], ones)
        return carry

    lax.fori_loop(0, VB, acc_body, 0)
    pltpu.sync_copy(deg_v, out_hbm.at[wid])


_deg_kernel = pl.kernel(
    _deg_body,
    out_type=jax.ShapeDtypeStruct((NW, EPW), _f32),
    mesh=_sc_mesh(),
    compiler_params=pltpu.CompilerParams(needs_layout_passes=False),
    scratch_types=[
        pltpu.VMEM((EPW,), jnp.int32),
        pltpu.VMEM((EPW,), _f32),
    ],
)


CHUNK = 256
NCHUNKS = 1280
N0 = 64
N1 = 16
NMAX = max(N0, N1)
NCHUNKS_ALLOC = 16 * N0 + 15 * N1 + NMAX
EPAD = max(NCHUNKS_ALLOC, NCHUNKS) * CHUNK


def _make_agg(d):
    def body(z_hbm, src_hbm, dst_hbm, out_hbm, src_v, dst_v,
             rows_a, rows_b, acc, sem_a, sem_b, sem_sa, sem_sb):
        c = lax.axis_index("c")
        s = lax.axis_index("s")
        row0 = s * ROWS_PER_TILE
        n_my = lax.select(c == 0, N0, N1)
        start = lax.select(c == 0, s * N0, 16 * N0 + s * N1)
        pltpu.sync_copy(src_hbm.at[pl.ds(start, NMAX)], src_v)
        pltpu.sync_copy(dst_hbm.at[pl.ds(start, NMAX)], dst_v)
        pltpu.sync_copy(z_hbm.at[pl.ds(row0, ROWS_PER_TILE)],
                        acc.at[pl.ds(row0, ROWS_PER_TILE)])
        plsc.subcore_barrier()

        pltpu.async_copy(z_hbm.at[src_v.at[0]], rows_a, sem_a)
        pltpu.async_copy(z_hbm.at[src_v.at[1]], rows_b, sem_b)

        def step(i, carry):
            j = 2 * i
            pltpu.make_async_copy(z_hbm.at[src_v.at[j]], rows_a, sem_a).wait()
            pltpu.async_copy(rows_a, acc.at[dst_v.at[j]], sem_sa, add=True)
            pltpu.make_async_copy(z_hbm.at[src_v.at[j + 1]], rows_b, sem_b).wait()
            pltpu.async_copy(rows_b, acc.at[dst_v.at[j + 1]], sem_sb, add=True)

            @pl.when(j + 2 < n_my)
            def _():
                pltpu.make_async_copy(rows_a, acc.at[dst_v.at[j]], sem_sa).wait()
                pltpu.async_copy(z_hbm.at[src_v.at[j + 2]], rows_a, sem_a)
                pltpu.make_async_copy(rows_b, acc.at[dst_v.at[j + 1]], sem_sb).wait()
                pltpu.async_copy(z_hbm.at[src_v.at[j + 3]], rows_b, sem_b)

            return carry

        lax.fori_loop(0, n_my // 2, step, 0)
        pltpu.make_async_copy(rows_a, acc.at[dst_v.at[0]], sem_sa).wait()
        pltpu.make_async_copy(rows_b, acc.at[dst_v.at[0]], sem_sb).wait()
        plsc.subcore_barrier()
        pltpu.sync_copy(acc.at[pl.ds(row0, ROWS_PER_TILE)],
                        out_hbm.at[c, pl.ds(row0, ROWS_PER_TILE)])

    return pl.kernel(
        body,
        out_type=jax.ShapeDtypeStruct((NC, NPAD, d), _f32),
        mesh=_sc_mesh(),
        compiler_params=pltpu.CompilerParams(use_tc_tiling_on_sc=False),
        scratch_types=[
            pltpu.VMEM((NMAX, CHUNK), jnp.int32),
            pltpu.VMEM((NMAX, CHUNK), jnp.int32),
            pltpu.VMEM((CHUNK, d), _f32),
            pltpu.VMEM((CHUNK, d), _f32),
            pltpu.VMEM_SHARED((NPAD, d), _f32),
            pltpu.SemaphoreType.DMA,
            pltpu.SemaphoreType.DMA,
            pltpu.SemaphoreType.DMA,
            pltpu.SemaphoreType.DMA,
        ],
    )


_agg_hid = _make_agg(D_HID)
_agg_out = _make_agg(D_OUT)


def _tc_a_body(deg_ref, x_ref, w_ref, z_ref, dinv_ref):
    deg_parts = deg_ref[...]
    ones = jnp.ones((NW, 1), _f32)
    deg = lax.dot_general(deg_parts, ones, (((0,), (0,)), ((), ())),
                          preferred_element_type=_f32)
    dinv = lax.rsqrt(deg + 1.0)
    dinv_pad = jnp.concatenate(
        [dinv, jnp.zeros((NPAD - N, 1), _f32)], axis=0)
    z = jnp.dot(x_ref[...], w_ref[...], preferred_element_type=_f32)
    z_ref[...] = z * dinv_pad
    dinv_ref[...] = dinv_pad


_tc_a = pl.pallas_call(
    _tc_a_body,
    out_shape=[
        jax.ShapeDtypeStruct((NPAD, D_HID), _f32),
        jax.ShapeDtypeStruct((NPAD, 1), _f32),
    ],
)


def _tc_b_body(p_ref, z1_ref, dinv_ref, b1_ref, w2_ref, z2_ref):
    agg = p_ref[0] + p_ref[1] - z1_ref[...]
    dinv = dinv_ref[...]
    h = jnp.maximum(dinv * agg + b1_ref[...], 0.0)
    z2 = jnp.dot(h, w2_ref[...], preferred_element_type=_f32)
    z2_ref[...] = z2 * dinv


_tc_b = pl.pallas_call(
    _tc_b_body,
    out_shape=jax.ShapeDtypeStruct((NPAD, D_OUT), _f32),
)


def _tc_c_body(q_ref, z2_ref, dinv_ref, b2_ref, out_ref):
    agg = q_ref[0] + q_ref[1] - z2_ref[...]
    o = dinv_ref[...] * agg + b2_ref[...]
    m = jnp.max(o, axis=1, keepdims=True)
    sh = o - m
    lse = jnp.log(jnp.sum(jnp.exp(sh), axis=1, keepdims=True))
    out_ref[...] = sh - lse


_tc_c = pl.pallas_call(
    _tc_c_body,
    out_shape=jax.ShapeDtypeStruct((NPAD, D_OUT), _f32),
)


def kernel(x, edge_index, W1, b1, W2, b2):
    src = edge_index[0]
    dst = edge_index[1]

    deg_parts = _deg_kernel(dst)

    x_pad = jnp.concatenate(
        [x, jnp.zeros((NPAD - N, D_IN), _f32)], axis=0)
    z1, dinv = _tc_a(deg_parts, x_pad, W1)

    pad_idx = jnp.full((EPAD - E,), N, jnp.int32)
    src_p = jnp.concatenate([src, pad_idx]).reshape(-1, CHUNK)
    dst_p = jnp.concatenate([dst, pad_idx]).reshape(-1, CHUNK)

    p = _agg_hid(z1, src_p, dst_p)
    z2 = _tc_b(p, z1, dinv, b1.reshape(1, D_HID), W2)
    q = _agg_out(z2, src_p, dst_p)
    out = _tc_c(q, z2, dinv, b2.reshape(1, D_OUT))
    return out[:N]


# speedup vs baseline: 1.5185x; 1.0189x over previous
import jax
import jax.numpy as jnp
from jax import lax
from jax.experimental import pallas as pl
from jax.experimental.pallas import tpu as pltpu
from jax.experimental.pallas import tpu_sc as plsc

N = 10000
E = 320000
D_IN = 128
D_HID = 64
D_OUT = 40

NC = 2
NS = 16
NW = NC * NS
NPAD = 10112
ROWS_PER_TILE = NPAD // NS

EPW = E // NW
VB = EPW // 16

_f32 = jnp.float32


def _sc_mesh():
    return plsc.VectorSubcoreMesh(core_axis_name="c", subcore_axis_name="s")


def _deg_body(dst_hbm, out_hbm, dst_v, deg_v):
    c = lax.axis_index("c")
    s = lax.axis_index("s")
    wid = s * NC + c
    pltpu.sync_copy(dst_hbm.at[pl.ds(wid * EPW, EPW)], dst_v)

    def zero_body(i, carry):
        deg_v[pl.ds(i * 16, 16)] = jnp.zeros((16,), _f32)
        return carry

    lax.fori_loop(0, EPW // 16, zero_body, 0)

    ones = jnp.full((16,), 1.0, _f32)

    def acc_body(i, carry):
        idx = dst_v[pl.ds(i * 16, 16)]
        plsc.addupdate_scatter(deg_v, [idx], ones)
        return carry

    lax.fori_loop(0, VB, acc_body, 0)
    pltpu.sync_copy(deg_v, out_hbm.at[wid])


_deg_kernel = pl.kernel(
    _deg_body,
    out_type=jax.ShapeDtypeStruct((NW, EPW), _f32),
    mesh=_sc_mesh(),
    compiler_params=pltpu.CompilerParams(needs_layout_passes=False),
    scratch_types=[
        pltpu.VMEM((EPW,), jnp.int32),
        pltpu.VMEM((EPW,), _f32),
    ],
)


CHUNK = 256
NCHUNKS = 1280
N0 = 72
N1 = 8
NMAX = max(N0, N1)
NCHUNKS_ALLOC = 16 * N0 + 15 * N1 + NMAX
EPAD = max(NCHUNKS_ALLOC, NCHUNKS) * CHUNK


def _make_agg(d):
    def body(z_hbm, src_hbm, dst_hbm, out_hbm, src_v, dst_v,
             rows_a, rows_b, acc, sem_a, sem_b, sem_sa, sem_sb):
        c = lax.axis_index("c")
        s = lax.axis_index("s")
        row0 = s * ROWS_PER_TILE
        n_my = lax.select(c == 0, N0, N1)
        start = lax.select(c == 0, s * N0, 16 * N0 + s * N1)
        pltpu.sync_copy(src_hbm.at[pl.ds(start, NMAX)], src_v)
        pltpu.sync_copy(dst_hbm.at[pl.ds(start, NMAX)], dst_v)
        pltpu.sync_copy(z_hbm.at[pl.ds(row0, ROWS_PER_TILE)],
                        acc.at[pl.ds(row0, ROWS_PER_TILE)])
        plsc.subcore_barrier()

        pltpu.async_copy(z_hbm.at[src_v.at[0]], rows_a, sem_a)
        pltpu.async_copy(z_hbm.at[src_v.at[1]], rows_b, sem_b)

        def step(i, carry):
            j = 2 * i
            pltpu.make_async_copy(z_hbm.at[src_v.at[j]], rows_a, sem_a).wait()
            pltpu.async_copy(rows_a, acc.at[dst_v.at[j]], sem_sa, add=True)
            pltpu.make_async_copy(z_hbm.at[src_v.at[j + 1]], rows_b, sem_b).wait()
            pltpu.async_copy(rows_b, acc.at[dst_v.at[j + 1]], sem_sb, add=True)

            @pl.when(j + 2 < n_my)
            def _():
                pltpu.make_async_copy(rows_a, acc.at[dst_v.at[j]], sem_sa).wait()
                pltpu.async_copy(z_hbm.at[src_v.at[j + 2]], rows_a, sem_a)
                pltpu.make_async_copy(rows_b, acc.at[dst_v.at[j + 1]], sem_sb).wait()
                pltpu.async_copy(z_hbm.at[src_v.at[j + 3]], rows_b, sem_b)

            return carry

        lax.fori_loop(0, n_my // 2, step, 0)
        pltpu.make_async_copy(rows_a, acc.at[dst_v.at[0]], sem_sa).wait()
        pltpu.make_async_copy(rows_b, acc.at[dst_v.at[0]], sem_sb).wait()
        plsc.subcore_barrier()
        pltpu.sync_copy(acc.at[pl.ds(row0, ROWS_PER_TILE)],
                        out_hbm.at[c, pl.ds(row0, ROWS_PER_TILE)])

    return pl.kernel(
        body,
        out_type=jax.ShapeDtypeStruct((NC, NPAD, d), _f32),
        mesh=_sc_mesh(),
        compiler_params=pltpu.CompilerParams(use_tc_tiling_on_sc=False),
        scratch_types=[
            pltpu.VMEM((NMAX, CHUNK), jnp.int32),
            pltpu.VMEM((NMAX, CHUNK), jnp.int32),
            pltpu.VMEM((CHUNK, d), _f32),
            pltpu.VMEM((CHUNK, d), _f32),
            pltpu.VMEM_SHARED((NPAD, d), _f32),
            pltpu.SemaphoreType.DMA,
            pltpu.SemaphoreType.DMA,
            pltpu.SemaphoreType.DMA,
            pltpu.SemaphoreType.DMA,
        ],
    )


_agg_hid = _make_agg(D_HID)
_agg_out = _make_agg(D_OUT)


def _tc_a_body(deg_ref, x_ref, w_ref, z_ref, dinv_ref):
    deg_parts = deg_ref[...]
    ones = jnp.ones((NW, 1), _f32)
    deg = lax.dot_general(deg_parts, ones, (((0,), (0,)), ((), ())),
                          preferred_element_type=_f32)
    dinv = lax.rsqrt(deg + 1.0)
    dinv_pad = jnp.concatenate(
        [dinv, jnp.zeros((NPAD - N, 1), _f32)], axis=0)
    z = jnp.dot(x_ref[...], w_ref[...], preferred_element_type=_f32)
    z_ref[...] = z * dinv_pad
    dinv_ref[...] = dinv_pad


_tc_a = pl.pallas_call(
    _tc_a_body,
    out_shape=[
        jax.ShapeDtypeStruct((NPAD, D_HID), _f32),
        jax.ShapeDtypeStruct((NPAD, 1), _f32),
    ],
)


def _tc_b_body(p_ref, z1_ref, dinv_ref, b1_ref, w2_ref, z2_ref):
    agg = p_ref[0] + p_ref[1] - z1_ref[...]
    dinv = dinv_ref[...]
    h = jnp.maximum(dinv * agg + b1_ref[...], 0.0)
    z2 = jnp.dot(h, w2_ref[...], preferred_element_type=_f32)
    z2_ref[...] = z2 * dinv


_tc_b = pl.pallas_call(
    _tc_b_body,
    out_shape=jax.ShapeDtypeStruct((NPAD, D_OUT), _f32),
)


def _tc_c_body(q_ref, z2_ref, dinv_ref, b2_ref, out_ref):
    agg = q_ref[0] + q_ref[1] - z2_ref[...]
    o = dinv_ref[...] * agg + b2_ref[...]
    m = jnp.max(o, axis=1, keepdims=True)
    sh = o - m
    lse = jnp.log(jnp.sum(jnp.exp(sh), axis=1, keepdims=True))
    out_ref[...] = sh - lse


_tc_c = pl.pallas_call(
    _tc_c_body,
    out_shape=jax.ShapeDtypeStruct((NPAD, D_OUT), _f32),
)


def kernel(x, edge_index, W1, b1, W2, b2):
    src = edge_index[0]
    dst = edge_index[1]

    deg_parts = _deg_kernel(dst)

    x_pad = jnp.concatenate(
        [x, jnp.zeros((NPAD - N, D_IN), _f32)], axis=0)
    z1, dinv = _tc_a(deg_parts, x_pad, W1)

    pad_idx = jnp.full((EPAD - E,), N, jnp.int32)
    src_p = jnp.concatenate([src, pad_idx]).reshape(-1, CHUNK)
    dst_p = jnp.concatenate([dst, pad_idx]).reshape(-1, CHUNK)

    p = _agg_hid(z1, src_p, dst_p)
    z2 = _tc_b(p, z1, dinv, b1.reshape(1, D_HID), W2)
    q = _agg_out(z2, src_p, dst_p)
    out = _tc_c(q, z2, dinv, b2.reshape(1, D_OUT))
    return out[:N]
